# baseline (speedup 1.0000x reference)
import functools

import jax
import jax.numpy as jnp
from jax import lax
from jax.experimental import pallas as pl
from jax.experimental.pallas import tpu as pltpu
from jax.experimental.pallas import tpu_sc as plsc

B = 4096
S = 200
H = 128
C = 1000

NC = 2
NS = 16
NW = NC * NS
LANES = 16
HCH = H // LANES

NBUF = 5

_mesh = plsc.VectorSubcoreMesh(core_axis_name="c", subcore_axis_name="s")


def _make_pool(nb, col_start):
    cols_per_tile = nb // NW

    @functools.partial(
        pl.kernel,
        mesh=_mesh,
        out_type=jax.ShapeDtypeStruct((nb, H), jnp.float32),
        scratch_types=(
            [pltpu.VMEM((S, cols_per_tile), jnp.int32)]
            + [pltpu.VMEM((cols_per_tile, H), jnp.float32)
               for _ in range(NBUF)]
            + [pltpu.VMEM((cols_per_tile, H), jnp.float32)]
            + [pltpu.SemaphoreType.DMA]
            + [pltpu.SemaphoreType.DMA for _ in range(NBUF)]
        ),
    )
    def pool(xt_hbm, emb_hbm, out_hbm, *refs):
        idxall = refs[0]
        rows = refs[1:1 + NBUF]
        acc = refs[1 + NBUF]
        sidx = refs[2 + NBUF]
        srows = refs[3 + NBUF:3 + 2 * NBUF]

        wid = lax.axis_index("s") * NC + lax.axis_index("c")
        col0 = col_start + wid * cols_per_tile

        pltpu.make_async_copy(
            xt_hbm.at[:, pl.ds(col0, cols_per_tile)], idxall, sidx).start()

        zero = jnp.zeros((LANES,), jnp.float32)

        def z_body(r, carry):
            for c in range(HCH):
                acc[r, pl.ds(c * LANES, LANES)] = zero
            return carry

        lax.fori_loop(0, cols_per_tile, z_body, 0)
        pltpu.make_async_copy(
            xt_hbm.at[:, pl.ds(col0, cols_per_tile)], idxall, sidx).wait()

        def gather(s, p):
            return pltpu.make_async_copy(
                emb_hbm.at[idxall.at[s]], rows[p], srows[p])

        for p in range(NBUF - 1):
            gather(p, p).start()

        AUNROLL = 4

        def accumulate(p):
            rv = rows[p]

            def a_body(ri, carry):
                r0 = ri * AUNROLL
                for dr in range(AUNROLL):
                    for c in range(HCH):
                        plsc.addupdate(
                            acc.at[r0 + dr, pl.ds(c * LANES, LANES)],
                            rv[r0 + dr, pl.ds(c * LANES, LANES)])
                return carry

            lax.fori_loop(0, cols_per_tile // AUNROLL, a_body, 0)

        def step(s, p):
            gather(s, p).wait()

            @pl.when(s + NBUF - 1 < S)
            def _():
                q = (p + NBUF - 1) % NBUF
                gather(s + NBUF - 1, q).start()
            accumulate(p)

        def body(gi, carry):
            for p in range(NBUF):
                step(gi * NBUF + p, p)
            return carry

        lax.fori_loop(0, S // NBUF, body, 0)

        def s_body(r, carry):
            for c in range(HCH):
                acc[r, pl.ds(c * LANES, LANES)] = (
                    acc[r, pl.ds(c * LANES, LANES)] * (1.0 / S))
            return carry

        lax.fori_loop(0, cols_per_tile, s_body, 0)
        pltpu.sync_copy(acc, out_hbm.at[pl.ds(col0, cols_per_tile)])

    return pool


_pool = _make_pool(B, 0)


def _mm_body_t(w_ref, bow_ref, b_ref, out_ref):
    out_ref[...] = (
        lax.dot_general(
            w_ref[...], bow_ref[...],
            (((1,), (1,)), ((), ())),
            preferred_element_type=jnp.float32,
        )
        + b_ref[...]
    )


BLK = 2048


def _matmul_t(bow, W, bcol):
    return pl.pallas_call(
        _mm_body_t,
        grid=(B // BLK,),
        in_specs=[
            pl.BlockSpec((C, H), lambda i: (0, 0)),
            pl.BlockSpec((BLK, H), lambda i: (i, 0)),
            pl.BlockSpec((C, 1), lambda i: (0, 0)),
        ],
        out_specs=pl.BlockSpec((C, BLK), lambda i: (0, i)),
        out_shape=jax.ShapeDtypeStruct((C, B), jnp.float32),
    )(W, bow, bcol)


def kernel(x, emb, W, b):
    xt = x.astype(jnp.int32).T
    bow = _pool(xt, emb)
    logits_t = _matmul_t(bow, W, b.reshape(C, 1))
    return logits_t.T


# speedup vs baseline: 1.4949x; 1.4949x over previous
import functools

import jax
import jax.numpy as jnp
from jax import lax
from jax.experimental import pallas as pl
from jax.experimental.pallas import tpu as pltpu
from jax.experimental.pallas import tpu_sc as plsc

B = 4096
S = 200
H = 128
C = 1000

NC = 2
NS = 16
NW = NC * NS
LANES = 16
HCH = H // LANES

NBUF = 5

_mesh = plsc.VectorSubcoreMesh(core_axis_name="c", subcore_axis_name="s")


def _make_pool(nb, col_start):
    cols_per_tile = nb // NW

    @functools.partial(
        pl.kernel,
        mesh=_mesh,
        out_type=jax.ShapeDtypeStruct((nb, H), jnp.float32),
        scratch_types=(
            [pltpu.VMEM((S, cols_per_tile), jnp.int32)]
            + [pltpu.VMEM((cols_per_tile, H), jnp.float32)
               for _ in range(NBUF)]
            + [pltpu.VMEM((cols_per_tile, H), jnp.float32)]
            + [pltpu.SemaphoreType.DMA]
            + [pltpu.SemaphoreType.DMA for _ in range(NBUF)]
        ),
    )
    def pool(xt_hbm, emb_hbm, out_hbm, *refs):
        idxall = refs[0]
        rows = refs[1:1 + NBUF]
        acc = refs[1 + NBUF]
        sidx = refs[2 + NBUF]
        srows = refs[3 + NBUF:3 + 2 * NBUF]

        wid = lax.axis_index("s") * NC + lax.axis_index("c")
        col0 = col_start + wid * cols_per_tile

        pltpu.make_async_copy(
            xt_hbm.at[:, pl.ds(col0, cols_per_tile)], idxall, sidx).start()

        zero = jnp.zeros((LANES,), jnp.float32)

        def z_body(r, carry):
            for c in range(HCH):
                acc[r, pl.ds(c * LANES, LANES)] = zero
            return carry

        lax.fori_loop(0, cols_per_tile, z_body, 0)
        pltpu.make_async_copy(
            xt_hbm.at[:, pl.ds(col0, cols_per_tile)], idxall, sidx).wait()

        def gather(s, p):
            return pltpu.make_async_copy(
                emb_hbm.at[idxall.at[s]], rows[p], srows[p])

        for p in range(NBUF - 1):
            gather(p, p).start()

        AUNROLL = 4

        def accumulate(p):
            rv = rows[p]

            def a_body(ri, carry):
                r0 = ri * AUNROLL
                for dr in range(AUNROLL):
                    for c in range(HCH):
                        acc[r0 + dr, pl.ds(c * LANES, LANES)] = (
                            rv[r0 + dr, pl.ds(c * LANES, LANES)])
                return carry

            lax.fori_loop(0, cols_per_tile // AUNROLL, a_body, 0)

        def step(s, p):
            gather(s, p).wait()

            @pl.when(s + NBUF - 1 < S)
            def _():
                q = (p + NBUF - 1) % NBUF
                gather(s + NBUF - 1, q).start()
            accumulate(p)

        def body(gi, carry):
            for p in range(NBUF):
                step(gi * NBUF + p, p)
            return carry

        lax.fori_loop(0, S // NBUF, body, 0)

        def s_body(r, carry):
            for c in range(HCH):
                acc[r, pl.ds(c * LANES, LANES)] = (
                    acc[r, pl.ds(c * LANES, LANES)] * (1.0 / S))
            return carry

        lax.fori_loop(0, cols_per_tile, s_body, 0)
        pltpu.sync_copy(acc, out_hbm.at[pl.ds(col0, cols_per_tile)])

    return pool


_pool = _make_pool(B, 0)


def _mm_body_t(w_ref, bow_ref, b_ref, out_ref):
    out_ref[...] = (
        lax.dot_general(
            w_ref[...], bow_ref[...],
            (((1,), (1,)), ((), ())),
            preferred_element_type=jnp.float32,
        )
        + b_ref[...]
    )


BLK = 2048


def _matmul_t(bow, W, bcol):
    return pl.pallas_call(
        _mm_body_t,
        grid=(B // BLK,),
        in_specs=[
            pl.BlockSpec((C, H), lambda i: (0, 0)),
            pl.BlockSpec((BLK, H), lambda i: (i, 0)),
            pl.BlockSpec((C, 1), lambda i: (0, 0)),
        ],
        out_specs=pl.BlockSpec((C, BLK), lambda i: (0, i)),
        out_shape=jax.ShapeDtypeStruct((C, B), jnp.float32),
    )(W, bow, bcol)


def kernel(x, emb, W, b):
    xt = x.astype(jnp.int32).T
    bow = _pool(xt, emb)
    logits_t = _matmul_t(bow, W, b.reshape(C, 1))
    return logits_t.T


# speedup vs baseline: 1.5345x; 1.0265x over previous
import functools

import jax
import jax.numpy as jnp
from jax import lax
from jax.experimental import pallas as pl
from jax.experimental.pallas import tpu as pltpu
from jax.experimental.pallas import tpu_sc as plsc

B = 4096
S = 200
H = 128
C = 1000

NC = 2
NS = 16
NW = NC * NS
LANES = 16
HCH = H // LANES

CH = 1
TOK = CH * S
NBUF = 4

_mesh = plsc.VectorSubcoreMesh(core_axis_name="c", subcore_axis_name="s")


def _make_pool(nb, row_start):
    rows_per_tile = nb // NW
    n_chunk = rows_per_tile // CH

    @functools.partial(
        pl.kernel,
        mesh=_mesh,
        out_type=jax.ShapeDtypeStruct((nb, H), jnp.float32),
        scratch_types=(
            [pltpu.VMEM((CH, TOK), jnp.int32) for _ in range(NBUF)]
            + [pltpu.VMEM((TOK, H), jnp.float32) for _ in range(NBUF)]
            + [pltpu.VMEM((rows_per_tile, H), jnp.float32)]
            + [pltpu.SemaphoreType.DMA for _ in range(2 * NBUF)]
        ),
    )
    def pool(x_hbm, emb_hbm, out_hbm, *refs):
        idx = refs[0:NBUF]
        rows = refs[NBUF:2 * NBUF]
        outst = refs[2 * NBUF]
        sidx = refs[2 * NBUF + 1:2 * NBUF + 1 + NBUF]
        srows = refs[2 * NBUF + 1 + NBUF:2 * NBUF + 1 + 2 * NBUF]

        wid = lax.axis_index("s") * NC + lax.axis_index("c")
        row0 = wid * rows_per_tile
        arow0 = row_start + row0

        def idx_copy(ci, p):
            return pltpu.make_async_copy(
                x_hbm.at[pl.ds(arow0 + ci * CH, CH), :], idx[p], sidx[p])

        def gather(p):
            return pltpu.make_async_copy(
                emb_hbm.at[idx[p].at[0]], rows[p], srows[p])

        def reduce_compute(ci, p):
            rv = rows[p]

            def s_body(si, accs):
                s = si * 2
                new = list(accs)
                for t in (s, s + 1):
                    for c in range(HCH):
                        new[c] = new[c] + rv[t, pl.ds(c * LANES, LANES)]
                return tuple(new)

            accs = lax.fori_loop(
                0, S // 2, s_body,
                tuple(jnp.zeros((LANES,), jnp.float32)
                      for _ in range(HCH)),
            )
            for c in range(HCH):
                outst[ci, pl.ds(c * LANES, LANES)] = accs[c] * (1.0 / S)

        for p in range(NBUF):
            idx_copy(p, p).start()
        for p in range(NBUF - 1):
            idx_copy(p, p).wait()
            gather(p).start()

        def step(ci, p):
            gather(p).wait()

            @pl.when(ci + NBUF < n_chunk)
            def _():
                idx_copy(ci + NBUF, p).start()

            @pl.when(ci + NBUF - 1 < n_chunk)
            def _():
                q = (p + NBUF - 1) % NBUF
                idx_copy(ci + NBUF - 1, q).wait()
                gather(q).start()
            reduce_compute(ci, p)

        def body(gi, carry):
            for p in range(NBUF):
                step(gi * NBUF + p, p)
            return carry

        lax.fori_loop(0, n_chunk // NBUF, body, 0)
        pltpu.sync_copy(outst, out_hbm.at[pl.ds(row0, rows_per_tile)])

    return pool


_pool = _make_pool(B, 0)


def _mm_body_t(w_ref, bow_ref, b_ref, out_ref):
    out_ref[...] = (
        lax.dot_general(
            w_ref[...], bow_ref[...],
            (((1,), (1,)), ((), ())),
            preferred_element_type=jnp.float32,
        )
        + b_ref[...]
    )


BLK = 2048


def _matmul_t(bow, W, bcol):
    return pl.pallas_call(
        _mm_body_t,
        grid=(B // BLK,),
        in_specs=[
            pl.BlockSpec((C, H), lambda i: (0, 0)),
            pl.BlockSpec((BLK, H), lambda i: (i, 0)),
            pl.BlockSpec((C, 1), lambda i: (0, 0)),
        ],
        out_specs=pl.BlockSpec((C, BLK), lambda i: (0, i)),
        out_shape=jax.ShapeDtypeStruct((C, B), jnp.float32),
    )(W, bow, bcol)


def kernel(x, emb, W, b):
    xf = x.astype(jnp.int32)
    bow = _pool(xf, emb)
    logits_t = _matmul_t(bow, W, b.reshape(C, 1))
    return logits_t.T
